# Initial kernel scaffold; baseline (speedup 1.0000x reference)
import jax
import jax.numpy as jnp
from jax.experimental import pallas as pl


def kernel(x, edge_index, W0, b0, W1, b1):
    raise NotImplementedError("write your pallas kernel here")


# speedup vs baseline: 7.9067x; 7.9067x over previous
import functools

import jax
import jax.numpy as jnp
from jax import lax
from jax.experimental import pallas as pl
from jax.experimental.pallas import tpu as pltpu
from jax.experimental.pallas import tpu_sc as plsc

NC = 2
NS = 16
CH = 128


def _mesh():
    return plsc.VectorSubcoreMesh(core_axis_name="c", subcore_axis_name="s")


def _make_deg_dinv(N, CPT, RPT):
    NPAD = NS * RPT
    RW = NPAD // (NC * NS)

    @functools.partial(
        pl.kernel,
        out_type=jax.ShapeDtypeStruct((NPAD,), jnp.float32),
        mesh=_mesh(),
        scratch_types=[
            pltpu.VMEM((CPT, CH), jnp.int32),
            pltpu.VMEM((CH,), jnp.float32),
            pltpu.VMEM((RW,), jnp.float32),
            pltpu.VMEM_SHARED((NPAD,), jnp.float32),
        ],
    )
    def deg_kernel(dst_hbm, zeros_hbm, dinv_hbm, idx_v, ones_v, dbuf, deg_sp):
        c = lax.axis_index("c")
        s = lax.axis_index("s")
        pltpu.sync_copy(zeros_hbm, deg_sp.at[pl.ds(s * RPT, RPT)])
        pltpu.sync_copy(dst_hbm.at[s], idx_v)
        for i in range(CH // 16):
            ones_v[pl.ds(i * 16, 16)] = jnp.ones((16,), jnp.float32)
        plsc.subcore_barrier()

        def step(j, carry):
            pltpu.sync_copy(ones_v, deg_sp.at[idx_v.at[j]], add=True)
            return carry

        lax.fori_loop(0, CPT, step, 0)
        plsc.subcore_barrier()

        wid = s * NC + c
        pltpu.sync_copy(deg_sp.at[pl.ds(wid * RW, RW)], dbuf)

        def rsqrt_step(i, carry):
            d = dbuf[pl.ds(i * 16, 16)] + 1.0
            di = lax.bitcast_convert_type(d, jnp.int32)
            yi = jnp.full((16,), 0x5F3759DF, jnp.int32) - (di >> 1)
            y = lax.bitcast_convert_type(yi, jnp.float32)
            for _ in range(3):
                y = y * (1.5 - 0.5 * d * y * y)
            dbuf[pl.ds(i * 16, 16)] = y
            return carry

        lax.fori_loop(0, RW // 16, rsqrt_step, 0)
        pltpu.sync_copy(dbuf, dinv_hbm.at[pl.ds(wid * RW, RW)])

    return deg_kernel


def _make_agg(N, D, CPT, RPT, CPG):
    NPAD = NS * RPT
    NG = CPT // CPG

    @functools.partial(
        pl.kernel,
        out_type=jax.ShapeDtypeStruct((NC, NPAD, D), jnp.float32),
        mesh=_mesh(),
        scratch_types=[
            pltpu.VMEM((CPG, CH), jnp.int32),
            pltpu.VMEM((CPG, CH), jnp.int32),
            pltpu.VMEM((CH, D), jnp.float32),
            pltpu.VMEM_SHARED((NPAD, D), jnp.float32),
        ],
    )
    def agg_kernel(table_hbm, src_hbm, dst_hbm, zeros_hbm, out_hbm,
                   src_v, dst_v, msg_v, acc_sp):
        c = lax.axis_index("c")
        s = lax.axis_index("s")
        pltpu.sync_copy(zeros_hbm, acc_sp.at[pl.ds(s * RPT, RPT)])
        plsc.subcore_barrier()

        def group(g, carry):
            pltpu.sync_copy(src_hbm.at[c, s, pl.ds(g * CPG, CPG)], src_v)
            pltpu.sync_copy(dst_hbm.at[c, s, pl.ds(g * CPG, CPG)], dst_v)

            def step(j, carry2):
                pltpu.sync_copy(table_hbm.at[src_v.at[j]], msg_v)
                pltpu.sync_copy(msg_v, acc_sp.at[dst_v.at[j]], add=True)
                return carry2

            lax.fori_loop(0, CPG, step, 0)
            return carry

        lax.fori_loop(0, NG, group, 0)
        plsc.subcore_barrier()
        pltpu.sync_copy(acc_sp.at[pl.ds(s * RPT, RPT)],
                        out_hbm.at[c, pl.ds(s * RPT, RPT)])

    return agg_kernel


def _tc_prescale(x, W, dinv2d, RB):
    N, Fin = x.shape
    H = W.shape[1]
    HH = H // NC
    NRB = N // RB

    def body(x_ref, w_ref, dinv_ref, o_ref):
        h = lax.dot_general(x_ref[...], w_ref[...], (((1,), (0,)), ((), ())),
                            precision=lax.Precision.HIGHEST,
                            preferred_element_type=jnp.float32)
        o_ref[...] = h * dinv_ref[...]

    return pl.pallas_call(
        body,
        grid=(NC, NRB),
        in_specs=[
            pl.BlockSpec((RB, Fin), lambda c, r: (r, 0)),
            pl.BlockSpec((Fin, HH), lambda c, r: (0, c)),
            pl.BlockSpec((RB, 1), lambda c, r: (r, 0)),
        ],
        out_specs=pl.BlockSpec((RB, HH), lambda c, r: (c * NRB + r, 0)),
        out_shape=jax.ShapeDtypeStruct((NC * N, HH), jnp.float32),
    )(x, W, dinv2d)


def _tc_mid(acc, hp, dinv2d, b2d, W, RB):
    _, _, HH = acc.shape
    N = hp.shape[0] // NC
    H = NC * HH
    Fout = W.shape[1]
    NRB = N // RB

    def body(accA, accB, hA, hB, dinv_ref, b_ref, w_ref, o_ref):
        agg = jnp.concatenate([accA[0], accB[0]], axis=1)
        hh = jnp.concatenate([hA[...], hB[...]], axis=1)
        z = jnp.maximum((agg + hh) * dinv_ref[...] + b_ref[...], 0.0)
        y = lax.dot_general(z, w_ref[...], (((1,), (0,)), ((), ())),
                            precision=lax.Precision.HIGHEST,
                            preferred_element_type=jnp.float32)
        o_ref[...] = y * dinv_ref[...]

    return pl.pallas_call(
        body,
        grid=(NRB,),
        in_specs=[
            pl.BlockSpec((1, RB, HH), lambda r: (0, r, 0)),
            pl.BlockSpec((1, RB, HH), lambda r: (1, r, 0)),
            pl.BlockSpec((RB, HH), lambda r: (r, 0)),
            pl.BlockSpec((RB, HH), lambda r: (NRB + r, 0)),
            pl.BlockSpec((RB, 1), lambda r: (r, 0)),
            pl.BlockSpec((1, H), lambda r: (0, 0)),
            pl.BlockSpec((H, Fout), lambda r: (0, 0)),
        ],
        out_specs=pl.BlockSpec((RB, Fout), lambda r: (r, 0)),
        out_shape=jax.ShapeDtypeStruct((N, Fout), jnp.float32),
    )(acc, acc, hp, hp, dinv2d, b2d, W)


def _tc_final(acc, hp, dinv2d, b2d, RB):
    _, _, Fout = acc.shape
    N = hp.shape[0]
    NRB = N // RB

    def body(accA, accB, h_ref, dinv_ref, b_ref, o_ref):
        agg = accA[0] + accB[0] + h_ref[...]
        o_ref[...] = jnp.maximum(agg * dinv_ref[...] + b_ref[...], 0.0)

    return pl.pallas_call(
        body,
        grid=(NRB,),
        in_specs=[
            pl.BlockSpec((1, RB, Fout), lambda r: (0, r, 0)),
            pl.BlockSpec((1, RB, Fout), lambda r: (1, r, 0)),
            pl.BlockSpec((RB, Fout), lambda r: (r, 0)),
            pl.BlockSpec((RB, 1), lambda r: (r, 0)),
            pl.BlockSpec((1, Fout), lambda r: (0, 0)),
        ],
        out_specs=pl.BlockSpec((RB, Fout), lambda r: (r, 0)),
        out_shape=jax.ShapeDtypeStruct((N, Fout), jnp.float32),
    )(acc, acc, hp, dinv2d, b2d)


def kernel(x, edge_index, W0, b0, W1, b1):
    N, Fin = x.shape
    H = W0.shape[1]
    Fout = W1.shape[1]
    E = edge_index.shape[1]
    RB = 400

    CPG = 16
    ept = -(-E // NS)
    CPT = -(-ept // (CH * CPG)) * CPG
    EPAD = NS * CPT * CH
    RPT = -(-(-(-N // NS)) // 16) * 16
    NPAD = NS * RPT

    ei = edge_index.astype(jnp.int32)
    src, dst = ei[0], ei[1]
    pad = EPAD - E
    srcp = jnp.concatenate([src, jnp.zeros((pad,), jnp.int32)])
    dstp = jnp.concatenate([dst, jnp.full((pad,), N, jnp.int32)])

    src_t = srcp.reshape(NS, CPT, CH)
    dst_t = dstp.reshape(NS, CPT, CH)
    src1 = jnp.stack([src_t, src_t + N])
    dst1 = jnp.stack([dst_t, dst_t])
    CPT2 = CPT // NC
    src2 = srcp.reshape(NC, NS, CPT2, CH)
    dst2 = dstp.reshape(NC, NS, CPT2, CH)

    zeros1 = jnp.zeros((RPT,), jnp.float32)
    zeros_h = jnp.zeros((RPT, H // NC), jnp.float32)
    zeros_f = jnp.zeros((RPT, Fout), jnp.float32)

    dinv_full = _make_deg_dinv(N, CPT, RPT)(dst_t, zeros1)
    dinv2d = dinv_full[:N].reshape(N, 1)

    h0p = _tc_prescale(x, W0, dinv2d, RB)
    acc0 = _make_agg(N, H // NC, CPT, RPT, CPG)(h0p, src1, dst1,
                                                zeros_h)
    h1p = _tc_mid(acc0, h0p, dinv2d, b0.reshape(1, H), W1, RB)
    acc1 = _make_agg(N, Fout, CPT2, RPT, CPG)(h1p, src2, dst2,
                                              zeros_f)
    return _tc_final(acc1, h1p, dinv2d, b1.reshape(1, Fout), RB)
